# Initial kernel scaffold; baseline (speedup 1.0000x reference)
import jax
import jax.numpy as jnp
from jax.experimental import pallas as pl


def kernel(actions, batch_time_shape, embedding, base_token):
    raise NotImplementedError("write your pallas kernel here")


# speedup vs baseline: 1.3148x; 1.3148x over previous
import functools

import jax
import jax.numpy as jnp
from jax import lax
from jax.experimental import pallas as pl
from jax.experimental.pallas import tpu as pltpu
from jax.experimental.pallas import tpu_sc as plsc


def _fold_base_into_table(embedding, base_bf):
    V, D = embedding.shape
    RB = 1000

    def body(emb_ref, base_ref, out_ref):
        out_ref[...] = emb_ref[...].astype(jnp.bfloat16) + base_ref[...]

    return pl.pallas_call(
        body,
        grid=(V // RB,),
        in_specs=[
            pl.BlockSpec((RB, D), lambda i: (i, 0)),
            pl.BlockSpec((1, D), lambda i: (0, 0)),
        ],
        out_specs=pl.BlockSpec((RB, D), lambda i: (i, 0)),
        out_shape=jax.ShapeDtypeStruct((V, D), jnp.bfloat16),
    )(embedding, base_bf.reshape(1, D))


def _sc_gather(tbl_i32, idx_flat, window):
    V, W = tbl_i32.shape
    N = idx_flat.shape[0]
    mesh = plsc.VectorSubcoreMesh(core_axis_name="core", subcore_axis_name="subcore")

    @functools.partial(
        pl.kernel,
        out_type=jax.ShapeDtypeStruct((N, W), jnp.int32),
        mesh=mesh,
        compiler_params=pltpu.CompilerParams(use_tc_tiling_on_sc=False),
    )
    def kern(tbl_hbm, idx_hbm, out_hbm):
        def body(idx_vmem, out_vmem):
            pltpu.sync_copy(tbl_hbm.at[idx_vmem.at[0]], out_vmem)

        pltpu.emit_pipeline(
            body,
            grid=(N // window,),
            in_specs=[pl.BlockSpec((1, window), index_map=lambda i: (0, i))],
            out_specs=[pl.BlockSpec((window, W), index_map=lambda i: (i, 0))],
            core_axis_name=("core", "subcore"),
            dimension_semantics=(pltpu.PARALLEL,),
        )(idx_hbm, out_hbm)

    return kern(tbl_i32, idx_flat.reshape(1, N))


def kernel(actions, batch_time_shape, embedding, base_token):
    V, D = embedding.shape
    B, L = actions.shape
    N = B * L
    W = D // 2

    base_bf = base_token.astype(jnp.bfloat16)
    tbl_bf = _fold_base_into_table(embedding, base_bf)
    tbl_i32 = lax.bitcast_convert_type(tbl_bf.reshape(V, W, 2), jnp.int32)

    out_i32 = _sc_gather(tbl_i32, actions.reshape(N), window=128)
    out_bf = lax.bitcast_convert_type(out_i32, jnp.bfloat16)
    return out_bf.reshape(B, L, D)


# speedup vs baseline: 1.3780x; 1.0481x over previous
import functools

import jax
import jax.numpy as jnp
from jax import lax
from jax.experimental import pallas as pl
from jax.experimental.pallas import tpu as pltpu
from jax.experimental.pallas import tpu_sc as plsc


def _fold_base_into_table(embedding, base_bf):
    V, D = embedding.shape
    RB = 1000

    def body(emb_ref, base_ref, out_ref):
        out_ref[...] = emb_ref[...].astype(jnp.bfloat16) + base_ref[...]

    return pl.pallas_call(
        body,
        grid=(V // RB,),
        in_specs=[
            pl.BlockSpec((RB, D), lambda i: (i, 0)),
            pl.BlockSpec((1, D), lambda i: (0, 0)),
        ],
        out_specs=pl.BlockSpec((RB, D), lambda i: (i, 0)),
        out_shape=jax.ShapeDtypeStruct((V, D), jnp.bfloat16),
    )(embedding, base_bf.reshape(1, D))


def _sc_gather(tbl_i32, idx_flat, window):
    V, W = tbl_i32.shape
    N = idx_flat.shape[0]
    mesh = plsc.VectorSubcoreMesh(core_axis_name="core", subcore_axis_name="subcore")

    @functools.partial(
        pl.kernel,
        out_type=jax.ShapeDtypeStruct((N, W), jnp.int32),
        mesh=mesh,
        compiler_params=pltpu.CompilerParams(use_tc_tiling_on_sc=False),
    )
    def kern(tbl_hbm, idx_hbm, out_hbm):
        def body(idx_vmem, out_vmem):
            pltpu.sync_copy(tbl_hbm.at[idx_vmem.at[0]], out_vmem)

        pltpu.emit_pipeline(
            body,
            grid=(N // window,),
            in_specs=[pl.BlockSpec((1, window), index_map=lambda i: (0, i))],
            out_specs=[pl.BlockSpec((window, W), index_map=lambda i: (i, 0))],
            core_axis_name=("core", "subcore"),
            dimension_semantics=(pltpu.PARALLEL,),
        )(idx_hbm, out_hbm)

    return kern(tbl_i32, idx_flat.reshape(1, N))


def kernel(actions, batch_time_shape, embedding, base_token):
    V, D = embedding.shape
    B, L = actions.shape
    N = B * L
    W = D // 2

    base_bf = base_token.astype(jnp.bfloat16)
    tbl_bf = _fold_base_into_table(embedding, base_bf)
    tbl_i32 = lax.bitcast_convert_type(tbl_bf.reshape(V, W, 2), jnp.int32)

    out_i32 = _sc_gather(tbl_i32, actions.reshape(N), window=1024)
    out_bf = lax.bitcast_convert_type(out_i32, jnp.bfloat16)
    return out_bf.reshape(B, L, D)


# speedup vs baseline: 2.5720x; 1.8664x over previous
import dataclasses
import functools

import jax
import jax.numpy as jnp
from jax import lax
from jax.experimental import pallas as pl
from jax.experimental.pallas import tpu as pltpu
from jax.experimental.pallas import tpu_sc as plsc

_FOLD_CHUNK = 4000


def _sc_compiler_params():
    cp = pltpu.CompilerParams(use_tc_tiling_on_sc=False)
    if "needs_layout_passes" in pltpu.CompilerParams.__dataclass_fields__:
        cp = dataclasses.replace(cp, needs_layout_passes=False)
    return cp


def _sc_fused(embT, idxT, base_token):
    D, V = embT.shape
    L, B = idxT.shape
    NBT = B // 128
    mesh = plsc.VectorSubcoreMesh(core_axis_name="core", subcore_axis_name="subcore")

    @functools.partial(
        pl.kernel,
        out_type=jax.ShapeDtypeStruct((L, D // 8, NBT, 4, 256), jnp.bfloat16),
        mesh=mesh,
        compiler_params=_sc_compiler_params(),
        scratch_types=[
            pltpu.VMEM((V,), jnp.int32),
            pltpu.VMEM((_FOLD_CHUNK,), jnp.float32),
            pltpu.VMEM((_FOLD_CHUNK,), jnp.float32),
            pltpu.VMEM((2, B), jnp.int32),
            pltpu.VMEM((2, NBT, 256), jnp.bfloat16),
            pltpu.VMEM((D,), jnp.float32),
            pltpu.SemaphoreType.DMA,
            pltpu.SemaphoreType.DMA,
            pltpu.SemaphoreType.DMA,
            pltpu.SemaphoreType.DMA,
            pltpu.SemaphoreType.DMA,
        ],
    )
    def kern(embT_hbm, idx_hbm, base_hbm, out_hbm,
             tbl_v, lo_v, hi_v, idx_v, out_v, base_v,
             s_i0, s_i1, s_o0, s_o1, s_f):
        core = lax.axis_index("core")
        sub = lax.axis_index("subcore")
        t = sub * 2 + core
        dt = t // 4
        sp = t % 4
        r0 = 2 * t

        pltpu.make_async_copy(base_hbm, base_v, s_f).start()
        pltpu.make_async_copy(base_hbm, base_v, s_f).wait()
        vlo = plsc.load_gather(base_v, [lax.broadcast(r0, (16,))])
        vhi = plsc.load_gather(base_v, [lax.broadcast(r0 + 1, (16,))])
        base_pair = plsc.pack(vlo, vhi, format=plsc.PackFormat.INTERLEAVED)

        @pl.loop(0, V // _FOLD_CHUNK)
        def _(k):
            off = k * _FOLD_CHUNK
            pltpu.make_async_copy(
                embT_hbm.at[r0, pl.ds(off, _FOLD_CHUNK)], lo_v, s_f).start()
            pltpu.make_async_copy(
                embT_hbm.at[r0 + 1, pl.ds(off, _FOLD_CHUNK)], hi_v, s_f).start()
            pltpu.make_async_copy(
                embT_hbm.at[r0, pl.ds(off, _FOLD_CHUNK)], lo_v, s_f).wait()
            pltpu.make_async_copy(
                embT_hbm.at[r0 + 1, pl.ds(off, _FOLD_CHUNK)], hi_v, s_f).wait()

            @pl.loop(0, _FOLD_CHUNK // 16)
            def _(i):
                a = lo_v[pl.ds(i * 16, 16)]
                b = hi_v[pl.ds(i * 16, 16)]
                pv = plsc.pack(a, b, format=plsc.PackFormat.INTERLEAVED) + base_pair
                tbl_v[pl.ds(off + i * 16, 16)] = plsc.bitcast(pv, jnp.int32)

        idx_sems = (s_i0, s_i1)
        out_sems = (s_o0, s_o1)

        def do_row(l, buf):
            nxt = 1 - buf

            @pl.when(l + 1 < L)
            def _():
                pltpu.make_async_copy(
                    idx_hbm.at[l + 1], idx_v.at[nxt], idx_sems[nxt]).start()

            pltpu.make_async_copy(
                idx_hbm.at[l], idx_v.at[buf], idx_sems[buf]).wait()

            @pl.when(l >= 2)
            def _():
                pltpu.make_async_copy(
                    out_v.at[buf], out_hbm.at[l - 2, dt, :, sp, :],
                    out_sems[buf]).wait()

            @pl.loop(0, NBT)
            def _(bt):
                for q in range(8):
                    iv = idx_v[buf, pl.ds(bt * 128 + q * 16, 16)]
                    w = plsc.load_gather(tbl_v, [iv])
                    out_v[buf, bt, pl.ds(q * 32, 32)] = plsc.bitcast(w, jnp.bfloat16)

            pltpu.make_async_copy(
                out_v.at[buf], out_hbm.at[l, dt, :, sp, :], out_sems[buf]).start()

        pltpu.make_async_copy(idx_hbm.at[0], idx_v.at[0], idx_sems[0]).start()

        @pl.loop(0, L, step=2)
        def _(l):
            do_row(l, 0)
            do_row(l + 1, 1)

        pltpu.make_async_copy(
            out_v.at[0], out_hbm.at[L - 2, dt, :, sp, :], out_sems[0]).wait()
        pltpu.make_async_copy(
            out_v.at[1], out_hbm.at[L - 1, dt, :, sp, :], out_sems[1]).wait()

    return kern(embT, idxT, base_token)


def kernel(actions, batch_time_shape, embedding, base_token):
    V, D = embedding.shape
    B, L = actions.shape

    out5 = _sc_fused(embedding.T, actions.T, base_token)
    return (
        out5.reshape(L, D // 8, B // 128, 4, 128, 2)
        .transpose(2, 4, 0, 1, 3, 5)
        .reshape(B, L, D)
    )


# speedup vs baseline: 3.7632x; 1.4632x over previous
import dataclasses
import functools

import jax
import jax.numpy as jnp
from jax import lax
from jax.experimental import pallas as pl
from jax.experimental.pallas import tpu as pltpu
from jax.experimental.pallas import tpu_sc as plsc

_FOLD_CHUNK = 4000


def _sc_compiler_params():
    cp = pltpu.CompilerParams(use_tc_tiling_on_sc=False)
    if "needs_layout_passes" in pltpu.CompilerParams.__dataclass_fields__:
        cp = dataclasses.replace(cp, needs_layout_passes=False)
    return cp


def _pair_halves(w0, w1, shift):
    if shift:
        w0 = lax.shift_right_logical(w0, shift)
        w1 = lax.shift_right_logical(w1, shift)
    packed = plsc.pack(w0, w1, format=plsc.PackFormat.INTERLEAVED)
    return plsc.bitcast(packed, jnp.bfloat16)


def _sc_fused(emb1, idx1, base_token, V, D, B, L):
    mesh = plsc.VectorSubcoreMesh(core_axis_name="core", subcore_axis_name="subcore")

    @functools.partial(
        pl.kernel,
        out_type=jax.ShapeDtypeStruct((L, D, B), jnp.bfloat16),
        mesh=mesh,
        compiler_params=_sc_compiler_params(),
        scratch_types=[
            pltpu.VMEM((V,), jnp.int32),
            pltpu.VMEM((_FOLD_CHUNK,), jnp.float32),
            pltpu.VMEM((_FOLD_CHUNK,), jnp.float32),
            pltpu.VMEM((2, B), jnp.int32),
            pltpu.VMEM((2, 2, B), jnp.bfloat16),
            pltpu.VMEM((D,), jnp.float32),
            pltpu.SemaphoreType.DMA,
            pltpu.SemaphoreType.DMA,
            pltpu.SemaphoreType.DMA,
            pltpu.SemaphoreType.DMA,
            pltpu.SemaphoreType.DMA,
        ],
    )
    def kern(emb_hbm, idx_hbm, base_hbm, out_hbm,
             tbl_v, lo_v, hi_v, idx_v, out_v, base_v,
             s_i0, s_i1, s_o0, s_o1, s_f):
        core = lax.axis_index("core")
        sub = lax.axis_index("subcore")
        t = sub * 2 + core
        r0 = 2 * t

        pltpu.make_async_copy(base_hbm, base_v, s_f).start()
        pltpu.make_async_copy(base_hbm, base_v, s_f).wait()
        vlo = plsc.load_gather(base_v, [lax.broadcast(r0, (16,))])
        vhi = plsc.load_gather(base_v, [lax.broadcast(r0 + 1, (16,))])
        base_pair = plsc.pack(vlo, vhi, format=plsc.PackFormat.INTERLEAVED)

        @pl.loop(0, V // _FOLD_CHUNK)
        def _(k):
            off = k * _FOLD_CHUNK
            pltpu.make_async_copy(
                emb_hbm.at[pl.ds(r0 * V + off, _FOLD_CHUNK)], lo_v, s_f).start()
            pltpu.make_async_copy(
                emb_hbm.at[pl.ds((r0 + 1) * V + off, _FOLD_CHUNK)], hi_v, s_f).start()
            pltpu.make_async_copy(
                emb_hbm.at[pl.ds(r0 * V + off, _FOLD_CHUNK)], lo_v, s_f).wait()
            pltpu.make_async_copy(
                emb_hbm.at[pl.ds((r0 + 1) * V + off, _FOLD_CHUNK)], hi_v, s_f).wait()

            @pl.loop(0, _FOLD_CHUNK // 16)
            def _(i):
                a = lo_v[pl.ds(i * 16, 16)]
                b = hi_v[pl.ds(i * 16, 16)]
                pv = plsc.pack(a, b, format=plsc.PackFormat.INTERLEAVED) + base_pair
                tbl_v[pl.ds(off + i * 16, 16)] = plsc.bitcast(pv, jnp.int32)

        idx_sems = (s_i0, s_i1)
        out_sems = (s_o0, s_o1)

        def do_row(l, buf):
            nxt = 1 - buf

            @pl.when(l + 1 < L)
            def _():
                pltpu.make_async_copy(
                    idx_hbm.at[pl.ds((l + 1) * B, B)], idx_v.at[nxt],
                    idx_sems[nxt]).start()

            pltpu.make_async_copy(
                idx_hbm.at[pl.ds(l * B, B)], idx_v.at[buf], idx_sems[buf]).wait()

            @pl.when(l >= 2)
            def _():
                pltpu.make_async_copy(
                    out_v.at[buf], out_hbm.at[l - 2, pl.ds(r0, 2), :],
                    out_sems[buf]).wait()

            @pl.loop(0, B // 128)
            def _(bt):
                for q in range(4):
                    goff = bt * 64 + q * 16
                    iv0 = idx_v[buf, pl.ds(goff, 16)]
                    iv1 = idx_v[buf, pl.ds(B // 2 + goff, 16)]
                    w0 = plsc.load_gather(tbl_v, [iv0])
                    w1 = plsc.load_gather(tbl_v, [iv1])
                    off = bt * 128 + q * 32
                    out_v[buf, 0, pl.ds(off, 32)] = _pair_halves(w0, w1, 0)
                    out_v[buf, 1, pl.ds(off, 32)] = _pair_halves(w0, w1, 16)

            pltpu.make_async_copy(
                out_v.at[buf], out_hbm.at[l, pl.ds(r0, 2), :], out_sems[buf]).start()

        pltpu.make_async_copy(
            idx_hbm.at[pl.ds(0, B)], idx_v.at[0], idx_sems[0]).start()

        @pl.loop(0, L, step=2)
        def _(l):
            do_row(l, 0)
            do_row(l + 1, 1)

        pltpu.make_async_copy(
            out_v.at[0], out_hbm.at[L - 2, pl.ds(r0, 2), :], out_sems[0]).wait()
        pltpu.make_async_copy(
            out_v.at[1], out_hbm.at[L - 1, pl.ds(r0, 2), :], out_sems[1]).wait()

    return kern(emb1, idx1, base_token)


def kernel(actions, batch_time_shape, embedding, base_token):
    V, D = embedding.shape
    B, L = actions.shape

    emb1 = embedding.T.reshape(V * D)
    idx1 = actions.T.reshape(L, B // 2, 2).transpose(0, 2, 1).reshape(B * L)
    out3 = _sc_fused(emb1, idx1, base_token, V, D, B, L)
    return out3.transpose(2, 0, 1)


# speedup vs baseline: 3.8009x; 1.0100x over previous
import dataclasses
import functools

import jax
import jax.numpy as jnp
from jax import lax
from jax.experimental import pallas as pl
from jax.experimental.pallas import tpu as pltpu
from jax.experimental.pallas import tpu_sc as plsc

_FOLD_CHUNK = 2000
_NIDX = 4


def _sc_compiler_params():
    cp = pltpu.CompilerParams(use_tc_tiling_on_sc=False)
    if "needs_layout_passes" in pltpu.CompilerParams.__dataclass_fields__:
        cp = dataclasses.replace(cp, needs_layout_passes=False)
    return cp


def _pair_halves(w0, w1, shift):
    if shift:
        w0 = lax.shift_right_logical(w0, shift)
        w1 = lax.shift_right_logical(w1, shift)
    packed = plsc.pack(w0, w1, format=plsc.PackFormat.INTERLEAVED)
    return plsc.bitcast(packed, jnp.bfloat16)


def _sc_fused(emb1, idx1, base_token, V, D, B, L):
    mesh = plsc.VectorSubcoreMesh(core_axis_name="core", subcore_axis_name="subcore")

    @functools.partial(
        pl.kernel,
        out_type=jax.ShapeDtypeStruct((L, D, B), jnp.bfloat16),
        mesh=mesh,
        compiler_params=_sc_compiler_params(),
        scratch_types=[
            pltpu.VMEM((V,), jnp.int32),
            pltpu.VMEM((_FOLD_CHUNK,), jnp.float32),
            pltpu.VMEM((_FOLD_CHUNK,), jnp.float32),
            pltpu.VMEM((_NIDX, B), jnp.int32),
            pltpu.VMEM((2, 2, B), jnp.bfloat16),
            pltpu.VMEM((D,), jnp.float32),
            pltpu.SemaphoreType.DMA,
            pltpu.SemaphoreType.DMA,
            pltpu.SemaphoreType.DMA,
            pltpu.SemaphoreType.DMA,
            pltpu.SemaphoreType.DMA,
            pltpu.SemaphoreType.DMA,
            pltpu.SemaphoreType.DMA,
        ],
    )
    def kern(emb_hbm, idx_hbm, base_hbm, out_hbm,
             tbl_v, lo_v, hi_v, idx_v, out_v, base_v,
             s_i0, s_i1, s_i2, s_i3, s_o0, s_o1, s_f):
        core = lax.axis_index("core")
        sub = lax.axis_index("subcore")
        t = sub * 2 + core
        r0 = 2 * t

        pltpu.make_async_copy(base_hbm, base_v, s_f).start()
        pltpu.make_async_copy(base_hbm, base_v, s_f).wait()
        vlo = plsc.load_gather(base_v, [lax.broadcast(r0, (16,))])
        vhi = plsc.load_gather(base_v, [lax.broadcast(r0 + 1, (16,))])
        base_pair = plsc.pack(vlo, vhi, format=plsc.PackFormat.INTERLEAVED)

        @pl.loop(0, V // _FOLD_CHUNK)
        def _(k):
            off = k * _FOLD_CHUNK
            pltpu.make_async_copy(
                emb_hbm.at[pl.ds(r0 * V + off, _FOLD_CHUNK)], lo_v, s_f).start()
            pltpu.make_async_copy(
                emb_hbm.at[pl.ds((r0 + 1) * V + off, _FOLD_CHUNK)], hi_v, s_f).start()
            pltpu.make_async_copy(
                emb_hbm.at[pl.ds(r0 * V + off, _FOLD_CHUNK)], lo_v, s_f).wait()
            pltpu.make_async_copy(
                emb_hbm.at[pl.ds((r0 + 1) * V + off, _FOLD_CHUNK)], hi_v, s_f).wait()

            @pl.loop(0, _FOLD_CHUNK // 16)
            def _(i):
                a = lo_v[pl.ds(i * 16, 16)]
                b = hi_v[pl.ds(i * 16, 16)]
                pv = plsc.pack(a, b, format=plsc.PackFormat.INTERLEAVED) + base_pair
                tbl_v[pl.ds(off + i * 16, 16)] = plsc.bitcast(pv, jnp.int32)

        idx_sems = (s_i0, s_i1, s_i2, s_i3)
        out_sems = (s_o0, s_o1)

        def do_row(l, j):
            pre = (j + _NIDX - 1) % _NIDX

            @pl.when(l + _NIDX - 1 < L)
            def _():
                pltpu.make_async_copy(
                    idx_hbm.at[pl.ds((l + _NIDX - 1) * B, B)], idx_v.at[pre],
                    idx_sems[pre]).start()

            pltpu.make_async_copy(
                idx_hbm.at[pl.ds(l * B, B)], idx_v.at[j], idx_sems[j]).wait()

            ob = j % 2

            @pl.when(l >= 2)
            def _():
                pltpu.make_async_copy(
                    out_v.at[ob], out_hbm.at[l - 2, pl.ds(r0, 2), :],
                    out_sems[ob]).wait()

            @pl.loop(0, B // 256)
            def _(bt):
                for q in range(8):
                    goff = bt * 128 + q * 16
                    iv0 = idx_v[j, pl.ds(goff, 16)]
                    iv1 = idx_v[j, pl.ds(B // 2 + goff, 16)]
                    w0 = plsc.load_gather(tbl_v, [iv0])
                    w1 = plsc.load_gather(tbl_v, [iv1])
                    off = bt * 256 + q * 32
                    out_v[ob, 0, pl.ds(off, 32)] = _pair_halves(w0, w1, 0)
                    out_v[ob, 1, pl.ds(off, 32)] = _pair_halves(w0, w1, 16)

            pltpu.make_async_copy(
                out_v.at[ob], out_hbm.at[l, pl.ds(r0, 2), :], out_sems[ob]).start()

        for j in range(_NIDX - 1):
            pltpu.make_async_copy(
                idx_hbm.at[pl.ds(j * B, B)], idx_v.at[j], idx_sems[j]).start()

        @pl.loop(0, L, step=_NIDX)
        def _(l):
            for j in range(_NIDX):
                do_row(l + j, j)

        pltpu.make_async_copy(
            out_v.at[0], out_hbm.at[L - 2, pl.ds(r0, 2), :], out_sems[0]).wait()
        pltpu.make_async_copy(
            out_v.at[1], out_hbm.at[L - 1, pl.ds(r0, 2), :], out_sems[1]).wait()

    return kern(emb1, idx1, base_token)


def kernel(actions, batch_time_shape, embedding, base_token):
    V, D = embedding.shape
    B, L = actions.shape

    emb1 = embedding.T.reshape(V * D)
    idx1 = actions.T.reshape(L, B // 2, 2).transpose(0, 2, 1).reshape(B * L)
    out3 = _sc_fused(emb1, idx1, base_token, V, D, B, L)
    return out3.transpose(2, 0, 1)


# speedup vs baseline: 4.3265x; 1.1383x over previous
import dataclasses
import functools

import jax
import jax.numpy as jnp
from jax import lax
from jax.experimental import pallas as pl
from jax.experimental.pallas import tpu as pltpu
from jax.experimental.pallas import tpu_sc as plsc

_FOLD_CHUNK = 2000
_NIDX = 4


def _sc_compiler_params():
    cp = pltpu.CompilerParams(use_tc_tiling_on_sc=False)
    if "needs_layout_passes" in pltpu.CompilerParams.__dataclass_fields__:
        cp = dataclasses.replace(cp, needs_layout_passes=False)
    return cp


def _pair_halves(w0, w1, shift):
    if shift:
        w0 = lax.shift_right_logical(w0, shift)
        w1 = lax.shift_right_logical(w1, shift)
    packed = plsc.pack(w0, w1, format=plsc.PackFormat.INTERLEAVED)
    return plsc.bitcast(packed, jnp.bfloat16)


def _sc_fused(emb1, idx1, base_token, V, D, B, L):
    mesh = plsc.VectorSubcoreMesh(core_axis_name="core", subcore_axis_name="subcore")

    @functools.partial(
        pl.kernel,
        out_type=jax.ShapeDtypeStruct((L, D, B), jnp.bfloat16),
        mesh=mesh,
        compiler_params=_sc_compiler_params(),
        scratch_types=[
            pltpu.VMEM((V,), jnp.int32),
            pltpu.VMEM((_FOLD_CHUNK,), jnp.float32),
            pltpu.VMEM((_FOLD_CHUNK,), jnp.float32),
            pltpu.VMEM((_NIDX, B), jnp.int32),
            pltpu.VMEM((2, 2, B), jnp.bfloat16),
            pltpu.VMEM((D,), jnp.float32),
            pltpu.SemaphoreType.DMA,
            pltpu.SemaphoreType.DMA,
            pltpu.SemaphoreType.DMA,
            pltpu.SemaphoreType.DMA,
            pltpu.SemaphoreType.DMA,
            pltpu.SemaphoreType.DMA,
            pltpu.SemaphoreType.DMA,
        ],
    )
    def kern(emb_hbm, idx_hbm, base_hbm, out_hbm,
             tbl_v, lo_v, hi_v, idx_v, out_v, base_v,
             s_i0, s_i1, s_i2, s_i3, s_o0, s_o1, s_f):
        core = lax.axis_index("core")
        sub = lax.axis_index("subcore")
        t = sub * 2 + core
        r0 = 2 * t

        pltpu.make_async_copy(base_hbm, base_v, s_f).start()
        pltpu.make_async_copy(base_hbm, base_v, s_f).wait()
        vlo = plsc.load_gather(base_v, [lax.broadcast(r0, (16,))])
        vhi = plsc.load_gather(base_v, [lax.broadcast(r0 + 1, (16,))])
        base_pair = plsc.pack(vlo, vhi, format=plsc.PackFormat.INTERLEAVED)

        @pl.loop(0, V // _FOLD_CHUNK)
        def _(k):
            off = k * _FOLD_CHUNK
            pltpu.make_async_copy(
                emb_hbm.at[pl.ds(r0 * V + off, _FOLD_CHUNK)], lo_v, s_f).start()
            pltpu.make_async_copy(
                emb_hbm.at[pl.ds((r0 + 1) * V + off, _FOLD_CHUNK)], hi_v, s_f).start()
            pltpu.make_async_copy(
                emb_hbm.at[pl.ds(r0 * V + off, _FOLD_CHUNK)], lo_v, s_f).wait()
            pltpu.make_async_copy(
                emb_hbm.at[pl.ds((r0 + 1) * V + off, _FOLD_CHUNK)], hi_v, s_f).wait()

            @plsc.parallel_loop(0, _FOLD_CHUNK // 16, unroll=4)
            def _(i):
                a = lo_v[pl.ds(i * 16, 16)]
                b = hi_v[pl.ds(i * 16, 16)]
                pv = plsc.pack(a, b, format=plsc.PackFormat.INTERLEAVED) + base_pair
                tbl_v[pl.ds(off + i * 16, 16)] = plsc.bitcast(pv, jnp.int32)

        idx_sems = (s_i0, s_i1, s_i2, s_i3)
        out_sems = (s_o0, s_o1)

        def do_row(l, j):
            pre = (j + _NIDX - 1) % _NIDX

            @pl.when(l + _NIDX - 1 < L)
            def _():
                pltpu.make_async_copy(
                    idx_hbm.at[pl.ds((l + _NIDX - 1) * B, B)], idx_v.at[pre],
                    idx_sems[pre]).start()

            pltpu.make_async_copy(
                idx_hbm.at[pl.ds(l * B, B)], idx_v.at[j], idx_sems[j]).wait()

            ob = j % 2

            @pl.when(l >= 2)
            def _():
                pltpu.make_async_copy(
                    out_v.at[ob], out_hbm.at[l - 2, pl.ds(r0, 2), :],
                    out_sems[ob]).wait()

            @plsc.parallel_loop(0, B // 256)
            def _(bt):
                for q in range(8):
                    goff = bt * 128 + q * 16
                    iv0 = idx_v[j, pl.ds(goff, 16)]
                    iv1 = idx_v[j, pl.ds(B // 2 + goff, 16)]
                    w0 = plsc.load_gather(tbl_v, [iv0])
                    w1 = plsc.load_gather(tbl_v, [iv1])
                    off = bt * 256 + q * 32
                    out_v[ob, 0, pl.ds(off, 32)] = _pair_halves(w0, w1, 0)
                    out_v[ob, 1, pl.ds(off, 32)] = _pair_halves(w0, w1, 16)

            pltpu.make_async_copy(
                out_v.at[ob], out_hbm.at[l, pl.ds(r0, 2), :], out_sems[ob]).start()

        for j in range(_NIDX - 1):
            pltpu.make_async_copy(
                idx_hbm.at[pl.ds(j * B, B)], idx_v.at[j], idx_sems[j]).start()

        @pl.loop(0, L, step=_NIDX)
        def _(l):
            for j in range(_NIDX):
                do_row(l + j, j)

        pltpu.make_async_copy(
            out_v.at[0], out_hbm.at[L - 2, pl.ds(r0, 2), :], out_sems[0]).wait()
        pltpu.make_async_copy(
            out_v.at[1], out_hbm.at[L - 1, pl.ds(r0, 2), :], out_sems[1]).wait()

    return kern(emb1, idx1, base_token)


def kernel(actions, batch_time_shape, embedding, base_token):
    V, D = embedding.shape
    B, L = actions.shape

    emb1 = embedding.T.reshape(V * D)
    idx1 = actions.T.reshape(L, B // 2, 2).transpose(0, 2, 1).reshape(B * L)
    out3 = _sc_fused(emb1, idx1, base_token, V, D, B, L)
    return out3.transpose(2, 0, 1)
